# baseline (speedup 1.0000x reference)
import functools
import numpy as np
import jax
import jax.numpy as jnp
from jax import lax
from jax.experimental import pallas as pl
from jax.experimental.pallas import tpu as pltpu
from jax.experimental.pallas import tpu_sc as plsc

N = 10000
NSTAR = 4
N4 = N + NSTAR
NP = 10240
HID = 128
HEADS = 8
HD = 16
E = 320000
EPAD = 327680
NW = 32
EW = EPAD // NW
C = 128
G = C // 16
SCALE = 0.25

_f32 = jnp.float32
_i32 = jnp.int32


def _headmask_M():
    ic = lax.broadcasted_iota(_i32, (HID, HEADS), 0)
    ih = lax.broadcasted_iota(_i32, (HID, HEADS), 1)
    return jnp.where(ic // HD == ih, 1.0, 0.0).astype(_f32)


def _repmat_R():
    ih = lax.broadcasted_iota(_i32, (HEADS, HID), 0)
    ic = lax.broadcasted_iota(_i32, (HEADS, HID), 1)
    return jnp.where(ic // HD == ih, 1.0, 0.0).astype(_f32)


def _ln_rows(x, s, b):
    m = jnp.mean(x, axis=-1, keepdims=True)
    v = jnp.mean((x - m) * (x - m), axis=-1, keepdims=True)
    return (x - m) / jnp.sqrt(v + 1e-5) * s + b


def _k1_body(x_b, wfl, bfl, h_out, seed_out):
    i = pl.program_id(0)
    hb = jnp.maximum(jnp.dot(x_b[...], wfl[...],
                             preferred_element_type=_f32) + bfl[...], 0.0)
    h_out[...] = hb
    sums = jnp.sum(hb, axis=0, keepdims=True)
    r8 = lax.broadcasted_iota(_i32, (8, HID), 0)
    blk = jnp.where(r8 == 0, sums, 0.0)

    @pl.when(i == 0)
    def _():
        seed_out[...] = jnp.zeros_like(seed_out)

    seed_out[...] += blk


def _k1(x, W_fl, b_fl):
    B = 2000
    return pl.pallas_call(
        _k1_body,
        grid=(N // B,),
        in_specs=[
            pl.BlockSpec((B, HID), lambda i: (i, 0)),
            pl.BlockSpec((HID, HID), lambda i: (0, 0)),
            pl.BlockSpec((1, HID), lambda i: (0, 0)),
        ],
        out_specs=[
            pl.BlockSpec((B, HID), lambda i: (i, 0)),
            pl.BlockSpec((8, HID), lambda i: (0, 0)),
        ],
        out_shape=[
            jax.ShapeDtypeStruct((N, HID), _f32),
            jax.ShapeDtypeStruct((8, HID), _f32),
        ],
    )(x, W_fl, b_fl.reshape(1, HID))


def _k2_body(h_b, seed, wq0, wk0, wv0, l0s, l0b, stars_out, num, den):
    i = pl.program_id(0)
    nsteps = pl.num_programs(0)

    @pl.when(i == 0)
    def _():
        num[...] = jnp.zeros_like(num)
        den[...] = jnp.zeros_like(den)

    q0 = jnp.dot(seed[0:1] * (1.0 / N), wq0[...],
                 preferred_element_type=_f32)
    kb = jnp.dot(h_b[...], wk0[...], preferred_element_type=_f32)
    vb = jnp.dot(h_b[...], wv0[...], preferred_element_type=_f32)
    inv = 1.0 / np.sqrt(HID)
    for s in range(NSTAR):
        sl = slice(s * HID, (s + 1) * HID)
        l = jnp.sum(kb[:, sl] * q0[:, sl], axis=1) * inv
        w = jnp.exp(l)
        nums = jnp.dot(w[None, :], vb[:, sl],
                       preferred_element_type=_f32)
        num[s:s + 1, :] += nums
        den[s:s + 1, :] += jnp.full((1, HID), 1.0) * jnp.sum(w)

    @pl.when(i == nsteps - 1)
    def _():
        st = num[...] / (den[...] + 1e-30)
        out = _ln_rows(st, l0s[...], l0b[...])
        r8 = lax.broadcasted_iota(_i32, (8, HID), 0)
        stars_out[...] = jnp.where(r8 < NSTAR, out, 0.0)


def _k2(h, seed, Wq0, Wk0, Wv0, ln0_s, ln0_b):
    B = 2000
    return pl.pallas_call(
        _k2_body,
        grid=(N // B,),
        in_specs=[
            pl.BlockSpec((B, HID), lambda i: (i, 0)),
            pl.BlockSpec((8, HID), lambda i: (0, 0)),
            pl.BlockSpec((HID, NSTAR * HID), lambda i: (0, 0)),
            pl.BlockSpec((HID, NSTAR * HID), lambda i: (0, 0)),
            pl.BlockSpec((HID, NSTAR * HID), lambda i: (0, 0)),
            pl.BlockSpec((1, HID), lambda i: (0, 0)),
            pl.BlockSpec((1, HID), lambda i: (0, 0)),
        ],
        out_specs=pl.BlockSpec((8, HID), lambda i: (0, 0)),
        out_shape=jax.ShapeDtypeStruct((8, HID), _f32),
        scratch_shapes=[
            pltpu.VMEM((8, HID), _f32),
            pltpu.VMEM((8, HID), _f32),
        ],
    )(h, seed, Wq0, Wk0, Wv0, ln0_s.reshape(1, HID), ln0_b.reshape(1, HID))


def _k3_body(x_b, wq, wk, wv, rel, q4_o, kr_o, v_o, wself_o):
    xb = x_b[...]
    q = jnp.dot(xb, wq[...], preferred_element_type=_f32) * SCALE
    k = jnp.dot(xb, wk[...], preferred_element_type=_f32) + rel[...]
    v = jnp.dot(xb, wv[...], preferred_element_type=_f32)
    M = _headmask_M()
    R = _repmat_R()
    s_h = jnp.dot(q * k, M, preferred_element_type=_f32)
    wrep = jnp.dot(jnp.exp(s_h), R, preferred_element_type=_f32)
    q4_o[...] = q
    kr_o[...] = k
    v_o[...] = v
    wself_o[...] = wrep


def _k3(x_all, Wq, Wk, Wv, rel):
    B = 640
    return pl.pallas_call(
        _k3_body,
        grid=(NP // B,),
        in_specs=[
            pl.BlockSpec((B, HID), lambda i: (i, 0)),
            pl.BlockSpec((HID, HID), lambda i: (0, 0)),
            pl.BlockSpec((HID, HID), lambda i: (0, 0)),
            pl.BlockSpec((HID, HID), lambda i: (0, 0)),
            pl.BlockSpec((1, HID), lambda i: (0, 0)),
        ],
        out_specs=[pl.BlockSpec((B, HID), lambda i: (i, 0))] * 4,
        out_shape=[jax.ShapeDtypeStruct((NP, HID), _f32)] * 4,
    )(x_all, Wq, Wk, Wv, rel.reshape(1, HID))


def _k4_body(kr_b, v_b, xstar, wq, acc_o, den_o, num, den):
    i = pl.program_id(0)
    nsteps = pl.num_programs(0)

    @pl.when(i == 0)
    def _():
        num[...] = jnp.zeros_like(num)
        den[...] = jnp.zeros_like(den)

    qs = jnp.dot(xstar[...], wq[...], preferred_element_type=_f32) * SCALE
    M = _headmask_M()
    R = _repmat_R()
    krb = kr_b[...]
    vb = v_b[...]
    for s in range(NSTAR):
        l = jnp.dot(krb * qs[s:s + 1], M, preferred_element_type=_f32)
        wrep = jnp.dot(jnp.exp(l), R, preferred_element_type=_f32)
        num[s:s + 1, :] += jnp.sum(wrep * vb, axis=0, keepdims=True)
        den[s:s + 1, :] += jnp.sum(wrep, axis=0, keepdims=True)

    @pl.when(i == nsteps - 1)
    def _():
        acc_o[...] = num[...]
        den_o[...] = den[...]


def _k4(kr, v, xstar, Wq):
    B = 400
    return pl.pallas_call(
        _k4_body,
        grid=(N // B,),
        in_specs=[
            pl.BlockSpec((B, HID), lambda i: (i, 0)),
            pl.BlockSpec((B, HID), lambda i: (i, 0)),
            pl.BlockSpec((8, HID), lambda i: (0, 0)),
            pl.BlockSpec((HID, HID), lambda i: (0, 0)),
        ],
        out_specs=[pl.BlockSpec((8, HID), lambda i: (0, 0))] * 2,
        out_shape=[jax.ShapeDtypeStruct((8, HID), _f32)] * 2,
        scratch_shapes=[
            pltpu.VMEM((8, HID), _f32),
            pltpu.VMEM((8, HID), _f32),
        ],
    )(kr, v, xstar, Wq)


def _k5_body(acc_b, denr_b, wself_b, v_b, x_b, sacc, sden,
             lns, lnb, out_b):
    i = pl.program_id(0)
    B = out_b.shape[0]
    grow = i * B + lax.broadcasted_iota(_i32, (B, HID), 0)
    mask_real = grow < N
    wself = jnp.where(mask_real, wself_b[...], 0.0)
    num = acc_b[...] + wself * v_b[...]
    den = denr_b[...] + wself
    for s in range(NSTAR):
        sel = grow == (N + s)
        num = num + jnp.where(sel, sacc[s:s + 1, :], 0.0)
        den = den + jnp.where(sel, sden[s:s + 1, :], 0.0)
    out = num / (den + 1e-16) + x_b[...]
    out_b[...] = _ln_rows(out, lns[...], lnb[...])


def _k5(acc, den_rep, wself, v, x_all, star_acc, star_den, lns, lnb):
    B = 640
    return pl.pallas_call(
        _k5_body,
        grid=(NP // B,),
        in_specs=[pl.BlockSpec((B, HID), lambda i: (i, 0))] * 5
        + [pl.BlockSpec((8, HID), lambda i: (0, 0))] * 2
        + [pl.BlockSpec((1, HID), lambda i: (0, 0))] * 2,
        out_specs=pl.BlockSpec((B, HID), lambda i: (i, 0)),
        out_shape=jax.ShapeDtypeStruct((NP, HID), _f32),
    )(acc, den_rep, wself, v, x_all, star_acc, star_den,
      lns.reshape(1, HID), lnb.reshape(1, HID))


def _k6_body(xn_b, stars, wq, wk, wv, lns, lnb, out_b, num, den):
    i = pl.program_id(0)
    nsteps = pl.num_programs(0)

    @pl.when(i == 0)
    def _():
        num[...] = jnp.zeros_like(num)
        den[...] = jnp.zeros_like(den)

    M = _headmask_M()
    R = _repmat_R()
    st = stars[...]
    qs = jnp.dot(st, wq[...], preferred_element_type=_f32) * SCALE
    ks = jnp.dot(xn_b[...], wk[...], preferred_element_type=_f32)
    vs = jnp.dot(xn_b[...], wv[...], preferred_element_type=_f32)
    for s in range(NSTAR):
        l = jnp.dot(ks * qs[s:s + 1], M, preferred_element_type=_f32)
        wrep = jnp.dot(jnp.exp(l), R, preferred_element_type=_f32)
        num[s:s + 1, :] += jnp.sum(wrep * vs, axis=0, keepdims=True)
        den[s:s + 1, :] += jnp.sum(wrep, axis=0, keepdims=True)

    @pl.when(i == nsteps - 1)
    def _():
        kst = jnp.dot(st, wk[...], preferred_element_type=_f32)
        vst = jnp.dot(st, wv[...], preferred_element_type=_f32)
        rkey = lax.broadcasted_iota(_i32, (8, HEADS), 0)
        nn = num[...]
        dd = den[...]
        for s in range(NSTAR):
            l = jnp.dot(kst * qs[s:s + 1], M, preferred_element_type=_f32)
            w = jnp.where(rkey < NSTAR, jnp.exp(l), 0.0)
            wrep = jnp.dot(w, R, preferred_element_type=_f32)
            rs = lax.broadcasted_iota(_i32, (8, HID), 0)
            upd_n = jnp.where(rs == s, jnp.sum(wrep * vst, axis=0,
                                               keepdims=True), 0.0)
            upd_d = jnp.where(rs == s, jnp.sum(wrep, axis=0, keepdims=True),
                              0.0)
            nn = nn + upd_n
            dd = dd + upd_d
        stv = nn / (dd + 1e-30) + st
        out = _ln_rows(stv, lns[...], lnb[...])
        r8 = lax.broadcasted_iota(_i32, (8, HID), 0)
        out_b[...] = jnp.where(r8 < NSTAR, out, 0.0)


def _k6(x_new, stars8, Wq, Wk, Wv, lns, lnb):
    B = 400
    return pl.pallas_call(
        _k6_body,
        grid=(N // B,),
        in_specs=[
            pl.BlockSpec((B, HID), lambda i: (i, 0)),
            pl.BlockSpec((8, HID), lambda i: (0, 0)),
            pl.BlockSpec((HID, HID), lambda i: (0, 0)),
            pl.BlockSpec((HID, HID), lambda i: (0, 0)),
            pl.BlockSpec((HID, HID), lambda i: (0, 0)),
            pl.BlockSpec((1, HID), lambda i: (0, 0)),
            pl.BlockSpec((1, HID), lambda i: (0, 0)),
        ],
        out_specs=pl.BlockSpec((8, HID), lambda i: (0, 0)),
        out_shape=jax.ShapeDtypeStruct((8, HID), _f32),
        scratch_shapes=[
            pltpu.VMEM((8, HID), _f32),
            pltpu.VMEM((8, HID), _f32),
        ],
    )(x_new, stars8, Wq, Wk, Wv, lns.reshape(1, HID), lnb.reshape(1, HID))


HW = 64
NH = 4
ET = EPAD // 16
NCH = ET // C


def _sc_edge_body(q_hbm, k_hbm, v_hbm, src_hbm, dst_hbm, acc_out, den_out,
                  siA, diA, dlA, qA, kA, vA, dA,
                  siB, diB, dlB, qB, kB, vB, dB,
                  acc_sh, den_sh, semAi, semA, semBi, semB):
    cid = lax.axis_index("c")
    sid = lax.axis_index("s")
    rows_per_tile = NP // 16

    zero16 = jnp.zeros((16,), _f32)

    def _zrow(r, _):
        for j in range(HW // 16):
            qA[r, pl.ds(j * 16, 16)] = zero16
        dA[r, pl.ds(0, 16)] = zero16
        dB[r, pl.ds(0, 16)] = zero16
        return 0

    lax.fori_loop(0, C, _zrow, 0)

    for t in range(rows_per_tile // C):
        pltpu.sync_copy(qA, acc_sh.at[pl.ds(sid * rows_per_tile + t * C, C)])
        pltpu.sync_copy(dA, den_sh.at[pl.ds(sid * rows_per_tile + t * C, C)])
    plsc.subcore_barrier()

    base = sid * NCH
    sets = ((siA, diA, dlA, qA, kA, vA, dA, semAi, semA),
            (siB, diB, dlB, qB, kB, vB, dB, semBi, semB))

    def _idx_issue(ch, s):
        si, di, dl, _, _, _, _, semi, _ = s
        pltpu.async_copy(src_hbm.at[cid, base + ch], si, semi)
        pltpu.async_copy(dst_hbm.at[cid, base + ch], di, semi)
        pltpu.async_copy(dst_hbm.at[0, base + ch], dl, semi)

    def _gather_issue(ch, s):
        si, di, _, q, k, v, _, semi, sem = s
        pltpu.make_async_copy(src_hbm.at[cid, base + ch], si, semi).wait()
        pltpu.make_async_copy(src_hbm.at[cid, base + ch], di, semi).wait()
        pltpu.make_async_copy(src_hbm.at[cid, base + ch], di, semi).wait()
        pltpu.async_copy(k_hbm.at[si], k, sem)
        pltpu.async_copy(v_hbm.at[si], v, sem)
        pltpu.async_copy(q_hbm.at[di], q, sem)

    def _gather_drain(s):
        si, di, _, q, k, v, _, _, sem = s
        pltpu.make_async_copy(k_hbm.at[si], k, sem).wait()
        pltpu.make_async_copy(v_hbm.at[si], v, sem).wait()
        pltpu.make_async_copy(q_hbm.at[di], q, sem).wait()

    def _compute(s):
        _, _, _, q, k, v, dd, _, _ = s

        def _group(g, _):
            rows = g * 16 + lax.iota(_i32, 16)
            iota = lax.iota(_i32, 16)
            for h in range(NH):
                cols = []
                prods = []
                for d in range(HD):
                    col = ((iota + d) & 15) + h * HD
                    cols.append(col)
                    qv = plsc.load_gather(q, [rows, col])
                    kv = plsc.load_gather(k, [rows, col])
                    prods.append(qv * kv)
                while len(prods) > 1:
                    prods = [a + b for a, b in zip(prods[::2], prods[1::2])]
                w = jnp.exp(prods[0])
                plsc.store_scatter(dd, [rows, jnp.full((16,), h, _i32)], w)
                for d in range(HD):
                    vv = plsc.load_gather(v, [rows, cols[d]])
                    plsc.store_scatter(q, [rows, cols[d]], w * vv)
            return 0

        lax.fori_loop(0, G, _group, 0)

    def _scatter(s):
        _, _, dl, q, _, _, dd, _, _ = s
        pltpu.sync_copy(q, acc_sh.at[dl], add=True)
        pltpu.sync_copy(dd, den_sh.at[dl], add=True)

    _idx_issue(0, sets[0])
    _gather_issue(0, sets[0])
    _idx_issue(1, sets[1])
    _gather_issue(1, sets[1])

    def _pair(t, _):
        ch0 = 2 * t
        _gather_drain(sets[0])
        _compute(sets[0])
        _scatter(sets[0])
        _idx_issue((ch0 + 2) % NCH, sets[0])
        _gather_issue((ch0 + 2) % NCH, sets[0])
        _gather_drain(sets[1])
        _compute(sets[1])
        _scatter(sets[1])
        _idx_issue((ch0 + 3) % NCH, sets[1])
        _gather_issue((ch0 + 3) % NCH, sets[1])
        return 0

    lax.fori_loop(0, NCH // 2, _pair, 0)
    _gather_drain(sets[0])
    _gather_drain(sets[1])
    plsc.subcore_barrier()

    r0 = sid * rows_per_tile
    pltpu.sync_copy(acc_sh.at[pl.ds(r0, rows_per_tile)],
                    acc_out.at[cid, pl.ds(r0, rows_per_tile)])
    pltpu.sync_copy(den_sh.at[pl.ds(r0, rows_per_tile)],
                    den_out.at[cid, pl.ds(r0, rows_per_tile)])


def _sc_edge(q2, k2, v2, src2, dst2):
    mesh = plsc.VectorSubcoreMesh(core_axis_name="c", subcore_axis_name="s")
    fn = pl.kernel(
        _sc_edge_body,
        mesh=mesh,
        compiler_params=pltpu.CompilerParams(use_tc_tiling_on_sc=False,
                                             needs_layout_passes=False),
        out_type=[
            jax.ShapeDtypeStruct((2, NP, HW), _f32),
            jax.ShapeDtypeStruct((2, NP, 16), _f32),
        ],
        scratch_types=[
            pltpu.VMEM((C,), _i32),
            pltpu.VMEM((C,), _i32),
            pltpu.VMEM((C,), _i32),
            pltpu.VMEM((C, HW), _f32),
            pltpu.VMEM((C, HW), _f32),
            pltpu.VMEM((C, HW), _f32),
            pltpu.VMEM((C, 16), _f32),
            pltpu.VMEM((C,), _i32),
            pltpu.VMEM((C,), _i32),
            pltpu.VMEM((C,), _i32),
            pltpu.VMEM((C, HW), _f32),
            pltpu.VMEM((C, HW), _f32),
            pltpu.VMEM((C, HW), _f32),
            pltpu.VMEM((C, 16), _f32),
            pltpu.VMEM_SHARED((NP, HW), _f32),
            pltpu.VMEM_SHARED((NP, 16), _f32),
            pltpu.SemaphoreType.DMA,
            pltpu.SemaphoreType.DMA,
            pltpu.SemaphoreType.DMA,
            pltpu.SemaphoreType.DMA,
        ],
    )
    return fn(q2, k2, v2, src2, dst2)


_SC_IMPL = _sc_edge


def kernel(x, edge_index, batch, W_fl, b_fl, Wq0, Wk0, Wv0, ln0_s, ln0_b,
           conv_Wq, conv_Wk, conv_Wv, conv_rel, conv_ln_s, conv_ln_b,
           star_Wq, star_Wk, star_Wv, star_ln_s, star_ln_b):
    h, seed = _k1(x, W_fl, b_fl)
    stars8 = _k2(h, seed, Wq0, Wk0, Wv0, ln0_s, ln0_b)

    pad_src = jnp.full((EPAD - E,), NP - 1, _i32)
    src = jnp.concatenate([edge_index[0].astype(_i32), pad_src])
    dst = jnp.concatenate([edge_index[1].astype(_i32), pad_src])
    src2 = jnp.stack([src, src + NP]).reshape(2, EPAD // C, C)
    dst2 = jnp.stack([dst, dst + NP]).reshape(2, EPAD // C, C)

    x_all = jnp.concatenate(
        [h, stars8[:NSTAR], jnp.zeros((NP - N4, HID), _f32)], axis=0)

    for i in range(3):
        q4, kr, v, wself = _k3(x_all, conv_Wq[i], conv_Wk[i], conv_Wv[i],
                               conv_rel[i, 0])
        q2 = jnp.concatenate([q4[:, :HW], q4[:, HW:]], axis=0)
        k2 = jnp.concatenate([kr[:, :HW], kr[:, HW:]], axis=0)
        v2 = jnp.concatenate([v[:, :HW], v[:, HW:]], axis=0)
        accs, dens = _SC_IMPL(q2, k2, v2, src2, dst2)
        acc = jnp.concatenate([accs[0], accs[1]], axis=1)
        den8 = jnp.concatenate([dens[0, :, :NH], dens[1, :, :NH]], axis=1)
        star_acc, star_den = _k4(kr, v, lax.dynamic_slice(x_all, (N, 0),
                                                          (8, HID)),
                                 conv_Wq[i])
        den_rep = jnp.repeat(den8, HD, axis=1)
        x_conv = _k5(acc, den_rep, wself, v, x_all,
                     star_acc, star_den, conv_ln_s[i], conv_ln_b[i])
        stars8 = _k6(x_conv[:N], stars8, star_Wq[i], star_Wk[i], star_Wv[i],
                     star_ln_s[i], star_ln_b[i])
        x_all = jnp.concatenate(
            [x_conv[:N], stars8[:NSTAR], jnp.zeros((NP - N4, HID), _f32)],
            axis=0)

    x_full = x_conv[:N4]
    stars = stars8[:NSTAR].reshape(1, NSTAR, HID)
    return (x_full, stars, x_full)


# speedup vs baseline: 1.5050x; 1.5050x over previous
import functools
import numpy as np
import jax
import jax.numpy as jnp
from jax import lax
from jax.experimental import pallas as pl
from jax.experimental.pallas import tpu as pltpu
from jax.experimental.pallas import tpu_sc as plsc

N = 10000
NSTAR = 4
N4 = N + NSTAR
NP = 10240
HID = 128
HEADS = 8
HD = 16
E = 320000
EPAD = 327680
NW = 32
EW = EPAD // NW
C = 128
G = C // 16
SCALE = 0.25

_f32 = jnp.float32
_i32 = jnp.int32


def _headmask_M():
    ic = lax.broadcasted_iota(_i32, (HID, HEADS), 0)
    ih = lax.broadcasted_iota(_i32, (HID, HEADS), 1)
    return jnp.where(ic // HD == ih, 1.0, 0.0).astype(_f32)


def _repmat_R():
    ih = lax.broadcasted_iota(_i32, (HEADS, HID), 0)
    ic = lax.broadcasted_iota(_i32, (HEADS, HID), 1)
    return jnp.where(ic // HD == ih, 1.0, 0.0).astype(_f32)


def _ln_rows(x, s, b):
    m = jnp.mean(x, axis=-1, keepdims=True)
    v = jnp.mean((x - m) * (x - m), axis=-1, keepdims=True)
    return (x - m) / jnp.sqrt(v + 1e-5) * s + b


def _k1_body(x_b, wfl, bfl, h_out, seed_out):
    i = pl.program_id(0)
    hb = jnp.maximum(jnp.dot(x_b[...], wfl[...],
                             preferred_element_type=_f32) + bfl[...], 0.0)
    h_out[...] = hb
    sums = jnp.sum(hb, axis=0, keepdims=True)
    r8 = lax.broadcasted_iota(_i32, (8, HID), 0)
    blk = jnp.where(r8 == 0, sums, 0.0)

    @pl.when(i == 0)
    def _():
        seed_out[...] = jnp.zeros_like(seed_out)

    seed_out[...] += blk


def _k1(x, W_fl, b_fl):
    B = 2000
    return pl.pallas_call(
        _k1_body,
        grid=(N // B,),
        in_specs=[
            pl.BlockSpec((B, HID), lambda i: (i, 0)),
            pl.BlockSpec((HID, HID), lambda i: (0, 0)),
            pl.BlockSpec((1, HID), lambda i: (0, 0)),
        ],
        out_specs=[
            pl.BlockSpec((B, HID), lambda i: (i, 0)),
            pl.BlockSpec((8, HID), lambda i: (0, 0)),
        ],
        out_shape=[
            jax.ShapeDtypeStruct((N, HID), _f32),
            jax.ShapeDtypeStruct((8, HID), _f32),
        ],
    )(x, W_fl, b_fl.reshape(1, HID))


def _k2_body(h_b, seed, wq0, wk0, wv0, l0s, l0b, stars_out, num, den):
    i = pl.program_id(0)
    nsteps = pl.num_programs(0)

    @pl.when(i == 0)
    def _():
        num[...] = jnp.zeros_like(num)
        den[...] = jnp.zeros_like(den)

    q0 = jnp.dot(seed[0:1] * (1.0 / N), wq0[...],
                 preferred_element_type=_f32)
    kb = jnp.dot(h_b[...], wk0[...], preferred_element_type=_f32)
    vb = jnp.dot(h_b[...], wv0[...], preferred_element_type=_f32)
    inv = 1.0 / np.sqrt(HID)
    for s in range(NSTAR):
        sl = slice(s * HID, (s + 1) * HID)
        l = jnp.sum(kb[:, sl] * q0[:, sl], axis=1) * inv
        w = jnp.exp(l)
        nums = jnp.dot(w[None, :], vb[:, sl],
                       preferred_element_type=_f32)
        num[s:s + 1, :] += nums
        den[s:s + 1, :] += jnp.full((1, HID), 1.0) * jnp.sum(w)

    @pl.when(i == nsteps - 1)
    def _():
        st = num[...] / (den[...] + 1e-30)
        out = _ln_rows(st, l0s[...], l0b[...])
        r8 = lax.broadcasted_iota(_i32, (8, HID), 0)
        stars_out[...] = jnp.where(r8 < NSTAR, out, 0.0)


def _k2(h, seed, Wq0, Wk0, Wv0, ln0_s, ln0_b):
    B = 2000
    return pl.pallas_call(
        _k2_body,
        grid=(N // B,),
        in_specs=[
            pl.BlockSpec((B, HID), lambda i: (i, 0)),
            pl.BlockSpec((8, HID), lambda i: (0, 0)),
            pl.BlockSpec((HID, NSTAR * HID), lambda i: (0, 0)),
            pl.BlockSpec((HID, NSTAR * HID), lambda i: (0, 0)),
            pl.BlockSpec((HID, NSTAR * HID), lambda i: (0, 0)),
            pl.BlockSpec((1, HID), lambda i: (0, 0)),
            pl.BlockSpec((1, HID), lambda i: (0, 0)),
        ],
        out_specs=pl.BlockSpec((8, HID), lambda i: (0, 0)),
        out_shape=jax.ShapeDtypeStruct((8, HID), _f32),
        scratch_shapes=[
            pltpu.VMEM((8, HID), _f32),
            pltpu.VMEM((8, HID), _f32),
        ],
    )(h, seed, Wq0, Wk0, Wv0, ln0_s.reshape(1, HID), ln0_b.reshape(1, HID))


def _k3_body(x_b, wq, wk, wv, rel, q4_o, kr_o, v_o, wself_o):
    xb = x_b[...]
    q = jnp.dot(xb, wq[...], preferred_element_type=_f32) * SCALE
    k = jnp.dot(xb, wk[...], preferred_element_type=_f32) + rel[...]
    v = jnp.dot(xb, wv[...], preferred_element_type=_f32)
    M = _headmask_M()
    R = _repmat_R()
    s_h = jnp.dot(q * k, M, preferred_element_type=_f32)
    wrep = jnp.dot(jnp.exp(s_h), R, preferred_element_type=_f32)
    q4_o[...] = q
    kr_o[...] = k
    v_o[...] = v
    wself_o[...] = wrep


def _k3(x_all, Wq, Wk, Wv, rel):
    B = 640
    return pl.pallas_call(
        _k3_body,
        grid=(NP // B,),
        in_specs=[
            pl.BlockSpec((B, HID), lambda i: (i, 0)),
            pl.BlockSpec((HID, HID), lambda i: (0, 0)),
            pl.BlockSpec((HID, HID), lambda i: (0, 0)),
            pl.BlockSpec((HID, HID), lambda i: (0, 0)),
            pl.BlockSpec((1, HID), lambda i: (0, 0)),
        ],
        out_specs=[pl.BlockSpec((B, HID), lambda i: (i, 0))] * 4,
        out_shape=[jax.ShapeDtypeStruct((NP, HID), _f32)] * 4,
    )(x_all, Wq, Wk, Wv, rel.reshape(1, HID))


def _k4_body(kr_b, v_b, xstar, wq, acc_o, den_o, num, den):
    i = pl.program_id(0)
    nsteps = pl.num_programs(0)

    @pl.when(i == 0)
    def _():
        num[...] = jnp.zeros_like(num)
        den[...] = jnp.zeros_like(den)

    qs = jnp.dot(xstar[...], wq[...], preferred_element_type=_f32) * SCALE
    M = _headmask_M()
    R = _repmat_R()
    krb = kr_b[...]
    vb = v_b[...]
    for s in range(NSTAR):
        l = jnp.dot(krb * qs[s:s + 1], M, preferred_element_type=_f32)
        wrep = jnp.dot(jnp.exp(l), R, preferred_element_type=_f32)
        num[s:s + 1, :] += jnp.sum(wrep * vb, axis=0, keepdims=True)
        den[s:s + 1, :] += jnp.sum(wrep, axis=0, keepdims=True)

    @pl.when(i == nsteps - 1)
    def _():
        acc_o[...] = num[...]
        den_o[...] = den[...]


def _k4(kr, v, xstar, Wq):
    B = 400
    return pl.pallas_call(
        _k4_body,
        grid=(N // B,),
        in_specs=[
            pl.BlockSpec((B, HID), lambda i: (i, 0)),
            pl.BlockSpec((B, HID), lambda i: (i, 0)),
            pl.BlockSpec((8, HID), lambda i: (0, 0)),
            pl.BlockSpec((HID, HID), lambda i: (0, 0)),
        ],
        out_specs=[pl.BlockSpec((8, HID), lambda i: (0, 0))] * 2,
        out_shape=[jax.ShapeDtypeStruct((8, HID), _f32)] * 2,
        scratch_shapes=[
            pltpu.VMEM((8, HID), _f32),
            pltpu.VMEM((8, HID), _f32),
        ],
    )(kr, v, xstar, Wq)


def _k5_body(acc_b, denr_b, wself_b, v_b, x_b, sacc, sden,
             lns, lnb, out_b):
    i = pl.program_id(0)
    B = out_b.shape[0]
    grow = i * B + lax.broadcasted_iota(_i32, (B, HID), 0)
    mask_real = grow < N
    wself = jnp.where(mask_real, wself_b[...], 0.0)
    num = acc_b[...] + wself * v_b[...]
    den = denr_b[...] + wself
    for s in range(NSTAR):
        sel = grow == (N + s)
        num = num + jnp.where(sel, sacc[s:s + 1, :], 0.0)
        den = den + jnp.where(sel, sden[s:s + 1, :], 0.0)
    out = num / (den + 1e-16) + x_b[...]
    out_b[...] = _ln_rows(out, lns[...], lnb[...])


def _k5(acc, den_rep, wself, v, x_all, star_acc, star_den, lns, lnb):
    B = 640
    return pl.pallas_call(
        _k5_body,
        grid=(NP // B,),
        in_specs=[pl.BlockSpec((B, HID), lambda i: (i, 0))] * 5
        + [pl.BlockSpec((8, HID), lambda i: (0, 0))] * 2
        + [pl.BlockSpec((1, HID), lambda i: (0, 0))] * 2,
        out_specs=pl.BlockSpec((B, HID), lambda i: (i, 0)),
        out_shape=jax.ShapeDtypeStruct((NP, HID), _f32),
    )(acc, den_rep, wself, v, x_all, star_acc, star_den,
      lns.reshape(1, HID), lnb.reshape(1, HID))


def _k6_body(xn_b, stars, wq, wk, wv, lns, lnb, out_b, num, den):
    i = pl.program_id(0)
    nsteps = pl.num_programs(0)

    @pl.when(i == 0)
    def _():
        num[...] = jnp.zeros_like(num)
        den[...] = jnp.zeros_like(den)

    M = _headmask_M()
    R = _repmat_R()
    st = stars[...]
    qs = jnp.dot(st, wq[...], preferred_element_type=_f32) * SCALE
    ks = jnp.dot(xn_b[...], wk[...], preferred_element_type=_f32)
    vs = jnp.dot(xn_b[...], wv[...], preferred_element_type=_f32)
    for s in range(NSTAR):
        l = jnp.dot(ks * qs[s:s + 1], M, preferred_element_type=_f32)
        wrep = jnp.dot(jnp.exp(l), R, preferred_element_type=_f32)
        num[s:s + 1, :] += jnp.sum(wrep * vs, axis=0, keepdims=True)
        den[s:s + 1, :] += jnp.sum(wrep, axis=0, keepdims=True)

    @pl.when(i == nsteps - 1)
    def _():
        kst = jnp.dot(st, wk[...], preferred_element_type=_f32)
        vst = jnp.dot(st, wv[...], preferred_element_type=_f32)
        rkey = lax.broadcasted_iota(_i32, (8, HEADS), 0)
        nn = num[...]
        dd = den[...]
        for s in range(NSTAR):
            l = jnp.dot(kst * qs[s:s + 1], M, preferred_element_type=_f32)
            w = jnp.where(rkey < NSTAR, jnp.exp(l), 0.0)
            wrep = jnp.dot(w, R, preferred_element_type=_f32)
            rs = lax.broadcasted_iota(_i32, (8, HID), 0)
            upd_n = jnp.where(rs == s, jnp.sum(wrep * vst, axis=0,
                                               keepdims=True), 0.0)
            upd_d = jnp.where(rs == s, jnp.sum(wrep, axis=0, keepdims=True),
                              0.0)
            nn = nn + upd_n
            dd = dd + upd_d
        stv = nn / (dd + 1e-30) + st
        out = _ln_rows(stv, lns[...], lnb[...])
        r8 = lax.broadcasted_iota(_i32, (8, HID), 0)
        out_b[...] = jnp.where(r8 < NSTAR, out, 0.0)


def _k6(x_new, stars8, Wq, Wk, Wv, lns, lnb):
    B = 400
    return pl.pallas_call(
        _k6_body,
        grid=(N // B,),
        in_specs=[
            pl.BlockSpec((B, HID), lambda i: (i, 0)),
            pl.BlockSpec((8, HID), lambda i: (0, 0)),
            pl.BlockSpec((HID, HID), lambda i: (0, 0)),
            pl.BlockSpec((HID, HID), lambda i: (0, 0)),
            pl.BlockSpec((HID, HID), lambda i: (0, 0)),
            pl.BlockSpec((1, HID), lambda i: (0, 0)),
            pl.BlockSpec((1, HID), lambda i: (0, 0)),
        ],
        out_specs=pl.BlockSpec((8, HID), lambda i: (0, 0)),
        out_shape=jax.ShapeDtypeStruct((8, HID), _f32),
        scratch_shapes=[
            pltpu.VMEM((8, HID), _f32),
            pltpu.VMEM((8, HID), _f32),
        ],
    )(x_new, stars8, Wq, Wk, Wv, lns.reshape(1, HID), lnb.reshape(1, HID))


HW = 64
NH = 4
ET = EPAD // 16
NCH = ET // C


def _sc_edge_body(q_hbm, k_hbm, v_hbm, src_hbm, dst_hbm, acc_out, den_out,
                  siA, diA, dlA, qA, kA, vA, dA,
                  siB, diB, dlB, qB, kB, vB, dB,
                  acc_sh, den_sh, semAi, semA, semBi, semB):
    cid = lax.axis_index("c")
    sid = lax.axis_index("s")
    rows_per_tile = NP // 16

    zero16 = jnp.zeros((16,), _f32)

    def _zrow(r, _):
        for j in range(HW // 16):
            qA[r, pl.ds(j * 16, 16)] = zero16
        dA[r, pl.ds(0, 16)] = zero16
        dB[r, pl.ds(0, 16)] = zero16
        return 0

    lax.fori_loop(0, C, _zrow, 0)

    for t in range(rows_per_tile // C):
        pltpu.sync_copy(qA, acc_sh.at[pl.ds(sid * rows_per_tile + t * C, C)])
        pltpu.sync_copy(dA, den_sh.at[pl.ds(sid * rows_per_tile + t * C, C)])
    plsc.subcore_barrier()

    base = sid * NCH
    sets = ((siA, diA, dlA, qA, kA, vA, dA, semAi, semA),
            (siB, diB, dlB, qB, kB, vB, dB, semBi, semB))

    def _idx_issue(ch, s):
        si, di, dl, _, _, _, _, semi, _ = s
        pltpu.async_copy(src_hbm.at[cid, base + ch], si, semi)
        pltpu.async_copy(dst_hbm.at[cid, base + ch], di, semi)
        pltpu.async_copy(dst_hbm.at[0, base + ch], dl, semi)

    def _gather_issue(ch, s):
        si, di, _, q, k, v, _, semi, sem = s
        pltpu.make_async_copy(src_hbm.at[cid, base + ch], si, semi).wait()
        pltpu.make_async_copy(src_hbm.at[cid, base + ch], di, semi).wait()
        pltpu.make_async_copy(src_hbm.at[cid, base + ch], di, semi).wait()
        pltpu.async_copy(k_hbm.at[si], k, sem)
        pltpu.async_copy(v_hbm.at[si], v, sem)
        pltpu.async_copy(q_hbm.at[di], q, sem)

    def _gather_drain(s):
        si, di, _, q, k, v, _, _, sem = s
        pltpu.make_async_copy(k_hbm.at[si], k, sem).wait()
        pltpu.make_async_copy(v_hbm.at[si], v, sem).wait()
        pltpu.make_async_copy(q_hbm.at[di], q, sem).wait()

    def _compute(s):
        _, _, _, q, k, v, dd, _, _ = s

        @functools.partial(plsc.parallel_loop, 0, G, unroll=2)
        def _group(g):
            rows = g * 16 + lax.iota(_i32, 16)
            iota = lax.iota(_i32, 16)
            for h in range(NH):
                cols = []
                l = jnp.zeros((16,), _f32)
                for d in range(HD):
                    col = ((iota + d) & 15) + h * HD
                    cols.append(col)
                    qv = plsc.load_gather(q, [rows, col])
                    kv = plsc.load_gather(k, [rows, col])
                    l = l + qv * kv
                w = jnp.exp(l)
                plsc.store_scatter(dd, [rows, jnp.full((16,), h, _i32)], w)
                for d in range(HD):
                    vv = plsc.load_gather(v, [rows, cols[d]])
                    plsc.store_scatter(q, [rows, cols[d]], w * vv)

    def _scatter(s):
        _, _, dl, q, _, _, dd, _, _ = s
        pltpu.sync_copy(q, acc_sh.at[dl], add=True)
        pltpu.sync_copy(dd, den_sh.at[dl], add=True)

    _idx_issue(0, sets[0])
    _gather_issue(0, sets[0])
    _idx_issue(1, sets[1])
    _gather_issue(1, sets[1])

    def _pair(t, _):
        ch0 = 2 * t
        _gather_drain(sets[0])
        _compute(sets[0])
        _scatter(sets[0])
        _idx_issue((ch0 + 2) % NCH, sets[0])
        _gather_issue((ch0 + 2) % NCH, sets[0])
        _gather_drain(sets[1])
        _compute(sets[1])
        _scatter(sets[1])
        _idx_issue((ch0 + 3) % NCH, sets[1])
        _gather_issue((ch0 + 3) % NCH, sets[1])
        return 0

    lax.fori_loop(0, NCH // 2, _pair, 0)
    _gather_drain(sets[0])
    _gather_drain(sets[1])
    plsc.subcore_barrier()

    r0 = sid * rows_per_tile
    pltpu.sync_copy(acc_sh.at[pl.ds(r0, rows_per_tile)],
                    acc_out.at[cid, pl.ds(r0, rows_per_tile)])
    pltpu.sync_copy(den_sh.at[pl.ds(r0, rows_per_tile)],
                    den_out.at[cid, pl.ds(r0, rows_per_tile)])


def _sc_edge(q2, k2, v2, src2, dst2):
    mesh = plsc.VectorSubcoreMesh(core_axis_name="c", subcore_axis_name="s")
    fn = pl.kernel(
        _sc_edge_body,
        mesh=mesh,
        compiler_params=pltpu.CompilerParams(use_tc_tiling_on_sc=False,
                                             needs_layout_passes=False),
        out_type=[
            jax.ShapeDtypeStruct((2, NP, HW), _f32),
            jax.ShapeDtypeStruct((2, NP, 16), _f32),
        ],
        scratch_types=[
            pltpu.VMEM((C,), _i32),
            pltpu.VMEM((C,), _i32),
            pltpu.VMEM((C,), _i32),
            pltpu.VMEM((C, HW), _f32),
            pltpu.VMEM((C, HW), _f32),
            pltpu.VMEM((C, HW), _f32),
            pltpu.VMEM((C, 16), _f32),
            pltpu.VMEM((C,), _i32),
            pltpu.VMEM((C,), _i32),
            pltpu.VMEM((C,), _i32),
            pltpu.VMEM((C, HW), _f32),
            pltpu.VMEM((C, HW), _f32),
            pltpu.VMEM((C, HW), _f32),
            pltpu.VMEM((C, 16), _f32),
            pltpu.VMEM_SHARED((NP, HW), _f32),
            pltpu.VMEM_SHARED((NP, 16), _f32),
            pltpu.SemaphoreType.DMA,
            pltpu.SemaphoreType.DMA,
            pltpu.SemaphoreType.DMA,
            pltpu.SemaphoreType.DMA,
        ],
    )
    return fn(q2, k2, v2, src2, dst2)


_SC_IMPL = _sc_edge


def kernel(x, edge_index, batch, W_fl, b_fl, Wq0, Wk0, Wv0, ln0_s, ln0_b,
           conv_Wq, conv_Wk, conv_Wv, conv_rel, conv_ln_s, conv_ln_b,
           star_Wq, star_Wk, star_Wv, star_ln_s, star_ln_b):
    h, seed = _k1(x, W_fl, b_fl)
    stars8 = _k2(h, seed, Wq0, Wk0, Wv0, ln0_s, ln0_b)

    pad_src = jnp.full((EPAD - E,), NP - 1, _i32)
    src = jnp.concatenate([edge_index[0].astype(_i32), pad_src])
    dst = jnp.concatenate([edge_index[1].astype(_i32), pad_src])
    src2 = jnp.stack([src, src + NP]).reshape(2, EPAD // C, C)
    dst2 = jnp.stack([dst, dst + NP]).reshape(2, EPAD // C, C)

    x_all = jnp.concatenate(
        [h, stars8[:NSTAR], jnp.zeros((NP - N4, HID), _f32)], axis=0)

    for i in range(3):
        q4, kr, v, wself = _k3(x_all, conv_Wq[i], conv_Wk[i], conv_Wv[i],
                               conv_rel[i, 0])
        q2 = jnp.concatenate([q4[:, :HW], q4[:, HW:]], axis=0)
        k2 = jnp.concatenate([kr[:, :HW], kr[:, HW:]], axis=0)
        v2 = jnp.concatenate([v[:, :HW], v[:, HW:]], axis=0)
        accs, dens = _SC_IMPL(q2, k2, v2, src2, dst2)
        acc = jnp.concatenate([accs[0], accs[1]], axis=1)
        den8 = jnp.concatenate([dens[0, :, :NH], dens[1, :, :NH]], axis=1)
        star_acc, star_den = _k4(kr, v, lax.dynamic_slice(x_all, (N, 0),
                                                          (8, HID)),
                                 conv_Wq[i])
        den_rep = jnp.repeat(den8, HD, axis=1)
        x_conv = _k5(acc, den_rep, wself, v, x_all,
                     star_acc, star_den, conv_ln_s[i], conv_ln_b[i])
        stars8 = _k6(x_conv[:N], stars8, star_Wq[i], star_Wk[i], star_Wv[i],
                     star_ln_s[i], star_ln_b[i])
        x_all = jnp.concatenate(
            [x_conv[:N], stars8[:NSTAR], jnp.zeros((NP - N4, HID), _f32)],
            axis=0)

    x_full = x_conv[:N4]
    stars = stars8[:NSTAR].reshape(1, NSTAR, HID)
    return (x_full, stars, x_full)


# speedup vs baseline: 1.5064x; 1.0009x over previous
import functools
import numpy as np
import jax
import jax.numpy as jnp
from jax import lax
from jax.experimental import pallas as pl
from jax.experimental.pallas import tpu as pltpu
from jax.experimental.pallas import tpu_sc as plsc

N = 10000
NSTAR = 4
N4 = N + NSTAR
NP = 10240
HID = 128
HEADS = 8
HD = 16
E = 320000
EPAD = 327680
NW = 32
EW = EPAD // NW
C = 128
G = C // 16
SCALE = 0.25

_f32 = jnp.float32
_i32 = jnp.int32


def _headmask_M():
    ic = lax.broadcasted_iota(_i32, (HID, HEADS), 0)
    ih = lax.broadcasted_iota(_i32, (HID, HEADS), 1)
    return jnp.where(ic // HD == ih, 1.0, 0.0).astype(_f32)


def _repmat_R():
    ih = lax.broadcasted_iota(_i32, (HEADS, HID), 0)
    ic = lax.broadcasted_iota(_i32, (HEADS, HID), 1)
    return jnp.where(ic // HD == ih, 1.0, 0.0).astype(_f32)


def _ln_rows(x, s, b):
    m = jnp.mean(x, axis=-1, keepdims=True)
    v = jnp.mean((x - m) * (x - m), axis=-1, keepdims=True)
    return (x - m) / jnp.sqrt(v + 1e-5) * s + b


def _k1_body(x_b, wfl, bfl, h_out, seed_out):
    i = pl.program_id(0)
    hb = jnp.maximum(jnp.dot(x_b[...], wfl[...],
                             preferred_element_type=_f32) + bfl[...], 0.0)
    h_out[...] = hb
    sums = jnp.sum(hb, axis=0, keepdims=True)
    r8 = lax.broadcasted_iota(_i32, (8, HID), 0)
    blk = jnp.where(r8 == 0, sums, 0.0)

    @pl.when(i == 0)
    def _():
        seed_out[...] = jnp.zeros_like(seed_out)

    seed_out[...] += blk


def _k1(x, W_fl, b_fl):
    B = 2000
    return pl.pallas_call(
        _k1_body,
        grid=(N // B,),
        in_specs=[
            pl.BlockSpec((B, HID), lambda i: (i, 0)),
            pl.BlockSpec((HID, HID), lambda i: (0, 0)),
            pl.BlockSpec((1, HID), lambda i: (0, 0)),
        ],
        out_specs=[
            pl.BlockSpec((B, HID), lambda i: (i, 0)),
            pl.BlockSpec((8, HID), lambda i: (0, 0)),
        ],
        out_shape=[
            jax.ShapeDtypeStruct((N, HID), _f32),
            jax.ShapeDtypeStruct((8, HID), _f32),
        ],
    )(x, W_fl, b_fl.reshape(1, HID))


def _k2_body(h_b, seed, wq0, wk0, wv0, l0s, l0b, stars_out, num, den):
    i = pl.program_id(0)
    nsteps = pl.num_programs(0)

    @pl.when(i == 0)
    def _():
        num[...] = jnp.zeros_like(num)
        den[...] = jnp.zeros_like(den)

    q0 = jnp.dot(seed[0:1] * (1.0 / N), wq0[...],
                 preferred_element_type=_f32)
    kb = jnp.dot(h_b[...], wk0[...], preferred_element_type=_f32)
    vb = jnp.dot(h_b[...], wv0[...], preferred_element_type=_f32)
    inv = 1.0 / np.sqrt(HID)
    for s in range(NSTAR):
        sl = slice(s * HID, (s + 1) * HID)
        l = jnp.sum(kb[:, sl] * q0[:, sl], axis=1) * inv
        w = jnp.exp(l)
        nums = jnp.dot(w[None, :], vb[:, sl],
                       preferred_element_type=_f32)
        num[s:s + 1, :] += nums
        den[s:s + 1, :] += jnp.full((1, HID), 1.0) * jnp.sum(w)

    @pl.when(i == nsteps - 1)
    def _():
        st = num[...] / (den[...] + 1e-30)
        out = _ln_rows(st, l0s[...], l0b[...])
        r8 = lax.broadcasted_iota(_i32, (8, HID), 0)
        stars_out[...] = jnp.where(r8 < NSTAR, out, 0.0)


def _k2(h, seed, Wq0, Wk0, Wv0, ln0_s, ln0_b):
    B = 2000
    return pl.pallas_call(
        _k2_body,
        grid=(N // B,),
        in_specs=[
            pl.BlockSpec((B, HID), lambda i: (i, 0)),
            pl.BlockSpec((8, HID), lambda i: (0, 0)),
            pl.BlockSpec((HID, NSTAR * HID), lambda i: (0, 0)),
            pl.BlockSpec((HID, NSTAR * HID), lambda i: (0, 0)),
            pl.BlockSpec((HID, NSTAR * HID), lambda i: (0, 0)),
            pl.BlockSpec((1, HID), lambda i: (0, 0)),
            pl.BlockSpec((1, HID), lambda i: (0, 0)),
        ],
        out_specs=pl.BlockSpec((8, HID), lambda i: (0, 0)),
        out_shape=jax.ShapeDtypeStruct((8, HID), _f32),
        scratch_shapes=[
            pltpu.VMEM((8, HID), _f32),
            pltpu.VMEM((8, HID), _f32),
        ],
    )(h, seed, Wq0, Wk0, Wv0, ln0_s.reshape(1, HID), ln0_b.reshape(1, HID))


def _k3_body(x_b, wq, wk, wv, rel, q4_o, kr_o, v_o, wself_o):
    xb = x_b[...]
    q = jnp.dot(xb, wq[...], preferred_element_type=_f32) * SCALE
    k = jnp.dot(xb, wk[...], preferred_element_type=_f32) + rel[...]
    v = jnp.dot(xb, wv[...], preferred_element_type=_f32)
    M = _headmask_M()
    R = _repmat_R()
    s_h = jnp.dot(q * k, M, preferred_element_type=_f32)
    wrep = jnp.dot(jnp.exp(s_h), R, preferred_element_type=_f32)
    q4_o[...] = q
    kr_o[...] = k
    v_o[...] = v
    wself_o[...] = wrep


def _k3(x_all, Wq, Wk, Wv, rel):
    B = 640
    return pl.pallas_call(
        _k3_body,
        grid=(NP // B,),
        in_specs=[
            pl.BlockSpec((B, HID), lambda i: (i, 0)),
            pl.BlockSpec((HID, HID), lambda i: (0, 0)),
            pl.BlockSpec((HID, HID), lambda i: (0, 0)),
            pl.BlockSpec((HID, HID), lambda i: (0, 0)),
            pl.BlockSpec((1, HID), lambda i: (0, 0)),
        ],
        out_specs=[pl.BlockSpec((B, HID), lambda i: (i, 0))] * 4,
        out_shape=[jax.ShapeDtypeStruct((NP, HID), _f32)] * 4,
    )(x_all, Wq, Wk, Wv, rel.reshape(1, HID))


def _k4_body(kr_b, v_b, xstar, wq, acc_o, den_o, num, den):
    i = pl.program_id(0)
    nsteps = pl.num_programs(0)

    @pl.when(i == 0)
    def _():
        num[...] = jnp.zeros_like(num)
        den[...] = jnp.zeros_like(den)

    qs = jnp.dot(xstar[...], wq[...], preferred_element_type=_f32) * SCALE
    M = _headmask_M()
    R = _repmat_R()
    krb = kr_b[...]
    vb = v_b[...]
    for s in range(NSTAR):
        l = jnp.dot(krb * qs[s:s + 1], M, preferred_element_type=_f32)
        wrep = jnp.dot(jnp.exp(l), R, preferred_element_type=_f32)
        num[s:s + 1, :] += jnp.sum(wrep * vb, axis=0, keepdims=True)
        den[s:s + 1, :] += jnp.sum(wrep, axis=0, keepdims=True)

    @pl.when(i == nsteps - 1)
    def _():
        acc_o[...] = num[...]
        den_o[...] = den[...]


def _k4(kr, v, xstar, Wq):
    B = 400
    return pl.pallas_call(
        _k4_body,
        grid=(N // B,),
        in_specs=[
            pl.BlockSpec((B, HID), lambda i: (i, 0)),
            pl.BlockSpec((B, HID), lambda i: (i, 0)),
            pl.BlockSpec((8, HID), lambda i: (0, 0)),
            pl.BlockSpec((HID, HID), lambda i: (0, 0)),
        ],
        out_specs=[pl.BlockSpec((8, HID), lambda i: (0, 0))] * 2,
        out_shape=[jax.ShapeDtypeStruct((8, HID), _f32)] * 2,
        scratch_shapes=[
            pltpu.VMEM((8, HID), _f32),
            pltpu.VMEM((8, HID), _f32),
        ],
    )(kr, v, xstar, Wq)


def _k5_body(acc_b, denr_b, wself_b, v_b, x_b, sacc, sden,
             lns, lnb, out_b):
    i = pl.program_id(0)
    B = out_b.shape[0]
    grow = i * B + lax.broadcasted_iota(_i32, (B, HID), 0)
    mask_real = grow < N
    wself = jnp.where(mask_real, wself_b[...], 0.0)
    num = acc_b[...] + wself * v_b[...]
    den = denr_b[...] + wself
    for s in range(NSTAR):
        sel = grow == (N + s)
        num = num + jnp.where(sel, sacc[s:s + 1, :], 0.0)
        den = den + jnp.where(sel, sden[s:s + 1, :], 0.0)
    out = num / (den + 1e-16) + x_b[...]
    out_b[...] = _ln_rows(out, lns[...], lnb[...])


def _k5(acc, den_rep, wself, v, x_all, star_acc, star_den, lns, lnb):
    B = 640
    return pl.pallas_call(
        _k5_body,
        grid=(NP // B,),
        in_specs=[pl.BlockSpec((B, HID), lambda i: (i, 0))] * 5
        + [pl.BlockSpec((8, HID), lambda i: (0, 0))] * 2
        + [pl.BlockSpec((1, HID), lambda i: (0, 0))] * 2,
        out_specs=pl.BlockSpec((B, HID), lambda i: (i, 0)),
        out_shape=jax.ShapeDtypeStruct((NP, HID), _f32),
    )(acc, den_rep, wself, v, x_all, star_acc, star_den,
      lns.reshape(1, HID), lnb.reshape(1, HID))


def _k6_body(xn_b, stars, wq, wk, wv, lns, lnb, out_b, num, den):
    i = pl.program_id(0)
    nsteps = pl.num_programs(0)

    @pl.when(i == 0)
    def _():
        num[...] = jnp.zeros_like(num)
        den[...] = jnp.zeros_like(den)

    M = _headmask_M()
    R = _repmat_R()
    st = stars[...]
    qs = jnp.dot(st, wq[...], preferred_element_type=_f32) * SCALE
    ks = jnp.dot(xn_b[...], wk[...], preferred_element_type=_f32)
    vs = jnp.dot(xn_b[...], wv[...], preferred_element_type=_f32)
    for s in range(NSTAR):
        l = jnp.dot(ks * qs[s:s + 1], M, preferred_element_type=_f32)
        wrep = jnp.dot(jnp.exp(l), R, preferred_element_type=_f32)
        num[s:s + 1, :] += jnp.sum(wrep * vs, axis=0, keepdims=True)
        den[s:s + 1, :] += jnp.sum(wrep, axis=0, keepdims=True)

    @pl.when(i == nsteps - 1)
    def _():
        kst = jnp.dot(st, wk[...], preferred_element_type=_f32)
        vst = jnp.dot(st, wv[...], preferred_element_type=_f32)
        rkey = lax.broadcasted_iota(_i32, (8, HEADS), 0)
        nn = num[...]
        dd = den[...]
        for s in range(NSTAR):
            l = jnp.dot(kst * qs[s:s + 1], M, preferred_element_type=_f32)
            w = jnp.where(rkey < NSTAR, jnp.exp(l), 0.0)
            wrep = jnp.dot(w, R, preferred_element_type=_f32)
            rs = lax.broadcasted_iota(_i32, (8, HID), 0)
            upd_n = jnp.where(rs == s, jnp.sum(wrep * vst, axis=0,
                                               keepdims=True), 0.0)
            upd_d = jnp.where(rs == s, jnp.sum(wrep, axis=0, keepdims=True),
                              0.0)
            nn = nn + upd_n
            dd = dd + upd_d
        stv = nn / (dd + 1e-30) + st
        out = _ln_rows(stv, lns[...], lnb[...])
        r8 = lax.broadcasted_iota(_i32, (8, HID), 0)
        out_b[...] = jnp.where(r8 < NSTAR, out, 0.0)


def _k6(x_new, stars8, Wq, Wk, Wv, lns, lnb):
    B = 400
    return pl.pallas_call(
        _k6_body,
        grid=(N // B,),
        in_specs=[
            pl.BlockSpec((B, HID), lambda i: (i, 0)),
            pl.BlockSpec((8, HID), lambda i: (0, 0)),
            pl.BlockSpec((HID, HID), lambda i: (0, 0)),
            pl.BlockSpec((HID, HID), lambda i: (0, 0)),
            pl.BlockSpec((HID, HID), lambda i: (0, 0)),
            pl.BlockSpec((1, HID), lambda i: (0, 0)),
            pl.BlockSpec((1, HID), lambda i: (0, 0)),
        ],
        out_specs=pl.BlockSpec((8, HID), lambda i: (0, 0)),
        out_shape=jax.ShapeDtypeStruct((8, HID), _f32),
        scratch_shapes=[
            pltpu.VMEM((8, HID), _f32),
            pltpu.VMEM((8, HID), _f32),
        ],
    )(x_new, stars8, Wq, Wk, Wv, lns.reshape(1, HID), lnb.reshape(1, HID))


HW = 64
NH = 4
ET = EPAD // 16
NCH = ET // C


def _sc_edge_body(q_hbm, k_hbm, v_hbm, src_hbm, dst_hbm, acc_out, den_out,
                  siA, diA, dlA, qA, kA, vA, dA, pA,
                  siB, diB, dlB, qB, kB, vB, dB, pB,
                  acc_sh, den_sh, semAi, semA, semBi, semB):
    cid = lax.axis_index("c")
    sid = lax.axis_index("s")
    rows_per_tile = NP // 16

    zero16 = jnp.zeros((16,), _f32)

    def _zrow(r, _):
        for j in range(HW // 16):
            qA[r, pl.ds(j * 16, 16)] = zero16
        dA[r, pl.ds(0, 16)] = zero16
        dB[r, pl.ds(0, 16)] = zero16
        return 0

    lax.fori_loop(0, C, _zrow, 0)

    for t in range(rows_per_tile // C):
        pltpu.sync_copy(qA, acc_sh.at[pl.ds(sid * rows_per_tile + t * C, C)])
        pltpu.sync_copy(dA, den_sh.at[pl.ds(sid * rows_per_tile + t * C, C)])
    plsc.subcore_barrier()

    base = sid * NCH
    sets = ((siA, diA, dlA, qA, kA, vA, dA, pA, semAi, semA),
            (siB, diB, dlB, qB, kB, vB, dB, pB, semBi, semB))

    def _idx_issue(ch, s):
        si, di, dl, _, _, _, _, _, semi, _ = s
        pltpu.async_copy(src_hbm.at[cid, base + ch], si, semi)
        pltpu.async_copy(dst_hbm.at[cid, base + ch], di, semi)
        pltpu.async_copy(dst_hbm.at[0, base + ch], dl, semi)

    def _gather_issue(ch, s):
        si, di, _, q, k, v, _, _, semi, sem = s
        pltpu.make_async_copy(src_hbm.at[cid, base + ch], si, semi).wait()
        pltpu.make_async_copy(src_hbm.at[cid, base + ch], di, semi).wait()
        pltpu.make_async_copy(src_hbm.at[cid, base + ch], di, semi).wait()
        pltpu.async_copy(k_hbm.at[si], k, sem)
        pltpu.async_copy(v_hbm.at[si], v, sem)
        pltpu.async_copy(q_hbm.at[di], q, sem)

    def _gather_drain(s):
        si, di, _, q, k, v, _, _, _, sem = s
        pltpu.make_async_copy(k_hbm.at[si], k, sem).wait()
        pltpu.make_async_copy(v_hbm.at[si], v, sem).wait()
        pltpu.make_async_copy(q_hbm.at[di], q, sem).wait()

    def _compute(s):
        _, _, _, q, k, v, dd, p, _, _ = s

        @functools.partial(plsc.parallel_loop, 0, G, unroll=2)
        def _group(g):
            rows = g * 16 + lax.iota(_i32, 16)
            iota = lax.iota(_i32, 16)
            for h in range(NH):
                cols = []
                l = jnp.zeros((16,), _f32)
                for d in range(HD):
                    col = ((iota + d) & 15) + h * HD
                    cols.append(col)
                    qv = plsc.load_gather(q, [rows, col])
                    kv = plsc.load_gather(k, [rows, col])
                    l = l + qv * kv
                w = jnp.exp(l)
                plsc.store_scatter(dd, [rows, jnp.full((16,), h, _i32)], w)
                for d in range(HD):
                    vv = plsc.load_gather(v, [rows, cols[d]])
                    plsc.store_scatter(p, [rows, cols[d]], w * vv)

    def _scatter(s):
        _, _, dl, _, _, _, dd, p, _, _ = s
        pltpu.sync_copy(p, acc_sh.at[dl], add=True)
        pltpu.sync_copy(dd, den_sh.at[dl], add=True)

    _idx_issue(0, sets[0])
    _gather_issue(0, sets[0])
    _idx_issue(1, sets[1])
    _gather_issue(1, sets[1])

    def _pair(t, _):
        ch0 = 2 * t
        _gather_drain(sets[0])
        _compute(sets[0])
        _scatter(sets[0])
        _idx_issue((ch0 + 2) % NCH, sets[0])
        _gather_issue((ch0 + 2) % NCH, sets[0])
        _gather_drain(sets[1])
        _compute(sets[1])
        _scatter(sets[1])
        _idx_issue((ch0 + 3) % NCH, sets[1])
        _gather_issue((ch0 + 3) % NCH, sets[1])
        return 0

    lax.fori_loop(0, NCH // 2, _pair, 0)
    _gather_drain(sets[0])
    _gather_drain(sets[1])
    plsc.subcore_barrier()

    r0 = sid * rows_per_tile
    pltpu.sync_copy(acc_sh.at[pl.ds(r0, rows_per_tile)],
                    acc_out.at[cid, pl.ds(r0, rows_per_tile)])
    pltpu.sync_copy(den_sh.at[pl.ds(r0, rows_per_tile)],
                    den_out.at[cid, pl.ds(r0, rows_per_tile)])


def _sc_edge(q2, k2, v2, src2, dst2):
    mesh = plsc.VectorSubcoreMesh(core_axis_name="c", subcore_axis_name="s")
    fn = pl.kernel(
        _sc_edge_body,
        mesh=mesh,
        compiler_params=pltpu.CompilerParams(use_tc_tiling_on_sc=False,
                                             needs_layout_passes=False),
        out_type=[
            jax.ShapeDtypeStruct((2, NP, HW), _f32),
            jax.ShapeDtypeStruct((2, NP, 16), _f32),
        ],
        scratch_types=[
            pltpu.VMEM((C,), _i32),
            pltpu.VMEM((C,), _i32),
            pltpu.VMEM((C,), _i32),
            pltpu.VMEM((C, HW), _f32),
            pltpu.VMEM((C, HW), _f32),
            pltpu.VMEM((C, HW), _f32),
            pltpu.VMEM((C, 16), _f32),
            pltpu.VMEM((C, HW), _f32),
            pltpu.VMEM((C,), _i32),
            pltpu.VMEM((C,), _i32),
            pltpu.VMEM((C,), _i32),
            pltpu.VMEM((C, HW), _f32),
            pltpu.VMEM((C, HW), _f32),
            pltpu.VMEM((C, HW), _f32),
            pltpu.VMEM((C, 16), _f32),
            pltpu.VMEM((C, HW), _f32),
            pltpu.VMEM_SHARED((NP, HW), _f32),
            pltpu.VMEM_SHARED((NP, 16), _f32),
            pltpu.SemaphoreType.DMA,
            pltpu.SemaphoreType.DMA,
            pltpu.SemaphoreType.DMA,
            pltpu.SemaphoreType.DMA,
        ],
    )
    return fn(q2, k2, v2, src2, dst2)


_SC_IMPL = _sc_edge


def kernel(x, edge_index, batch, W_fl, b_fl, Wq0, Wk0, Wv0, ln0_s, ln0_b,
           conv_Wq, conv_Wk, conv_Wv, conv_rel, conv_ln_s, conv_ln_b,
           star_Wq, star_Wk, star_Wv, star_ln_s, star_ln_b):
    h, seed = _k1(x, W_fl, b_fl)
    stars8 = _k2(h, seed, Wq0, Wk0, Wv0, ln0_s, ln0_b)

    pad_src = jnp.full((EPAD - E,), NP - 1, _i32)
    src = jnp.concatenate([edge_index[0].astype(_i32), pad_src])
    dst = jnp.concatenate([edge_index[1].astype(_i32), pad_src])
    src2 = jnp.stack([src, src + NP]).reshape(2, EPAD // C, C)
    dst2 = jnp.stack([dst, dst + NP]).reshape(2, EPAD // C, C)

    x_all = jnp.concatenate(
        [h, stars8[:NSTAR], jnp.zeros((NP - N4, HID), _f32)], axis=0)

    for i in range(3):
        q4, kr, v, wself = _k3(x_all, conv_Wq[i], conv_Wk[i], conv_Wv[i],
                               conv_rel[i, 0])
        q2 = jnp.concatenate([q4[:, :HW], q4[:, HW:]], axis=0)
        k2 = jnp.concatenate([kr[:, :HW], kr[:, HW:]], axis=0)
        v2 = jnp.concatenate([v[:, :HW], v[:, HW:]], axis=0)
        accs, dens = _SC_IMPL(q2, k2, v2, src2, dst2)
        acc = jnp.concatenate([accs[0], accs[1]], axis=1)
        den8 = jnp.concatenate([dens[0, :, :NH], dens[1, :, :NH]], axis=1)
        star_acc, star_den = _k4(kr, v, lax.dynamic_slice(x_all, (N, 0),
                                                          (8, HID)),
                                 conv_Wq[i])
        den_rep = jnp.repeat(den8, HD, axis=1)
        x_conv = _k5(acc, den_rep, wself, v, x_all,
                     star_acc, star_den, conv_ln_s[i], conv_ln_b[i])
        stars8 = _k6(x_conv[:N], stars8, star_Wq[i], star_Wk[i], star_Wv[i],
                     star_ln_s[i], star_ln_b[i])
        x_all = jnp.concatenate(
            [x_conv[:N], stars8[:NSTAR], jnp.zeros((NP - N4, HID), _f32)],
            axis=0)

    x_full = x_conv[:N4]
    stars = stars8[:NSTAR].reshape(1, NSTAR, HID)
    return (x_full, stars, x_full)
